# baseline (speedup 1.0000x reference)
import functools

import jax
import jax.numpy as jnp
from jax import lax
from jax.experimental import pallas as pl
from jax.experimental.pallas import tpu as pltpu
from jax.experimental.pallas import tpu_sc as plsc

_NT = 7300
_NG = 2000
_DAYS = 365
_NY = _NT // _DAYS
_NGTC = 1232
_GTC = 112
_GB = 8
_GG = 4
_NTASK = (_NG - _NGTC) // _GB
_NWORK = 32
_TPW = -(-_NTASK // _NWORK)
_SPLIT = 3712
_LA = _SPLIT
_LB = _NT - _SPLIT
_YSPLIT = _SPLIT // _DAYS
_L10A = _SPLIT - _YSPLIT * _DAYS
_L10B = _DAYS - _L10A
_LB1 = ((_NT // 128) * 128) - _SPLIT
_NTAIL = _NT - _SPLIT - _LB1
_TAILW = 128
_LBPAD = _LB1 + _TAILW


def _sc_partials(o3, t3, otail, ttail):
    mesh = plsc.VectorSubcoreMesh(core_axis_name="c", subcore_axis_name="s")

    @functools.partial(
        pl.kernel,
        out_type=jax.ShapeDtypeStruct((_NWORK, 8, 16), jnp.float32),
        mesh=mesh,
        scratch_types=[
            pltpu.VMEM((_GB, _LA), jnp.float32),
            pltpu.VMEM((_GB, _LA), jnp.float32),
            pltpu.VMEM((_GB, _LBPAD), jnp.float32),
            pltpu.VMEM((_GB, _LBPAD), jnp.float32),
            pltpu.VMEM((8, 16), jnp.float32),
            pltpu.SMEM((_GB,), jnp.float32),
            pltpu.SemaphoreType.DMA,
            pltpu.SemaphoreType.DMA,
            pltpu.SemaphoreType.DMA,
            pltpu.SemaphoreType.DMA,
        ],
        compiler_params=pltpu.CompilerParams(needs_layout_passes=False),
    )
    def body(o_hbm, t_hbm, otail_hbm, ttail_hbm, out_hbm, pa, ta, pb, tb,
             obuf, y10s, sa0, sa1, sb0, sb1):
        wid = lax.axis_index("s") * 2 + lax.axis_index("c")
        lane = lax.broadcasted_iota(jnp.int32, (16,), 0)
        zero16 = jnp.zeros((16,), jnp.float32)

        def g0_of(k):
            return pl.multiple_of(
                jnp.minimum(wid + _NWORK * k, _NTASK - 1) * _GB + _NGTC, _GB)

        def copies_a(k):
            g0 = g0_of(k)
            return (pltpu.make_async_copy(
                        o_hbm.at[0, pl.ds(g0, _GB), pl.ds(0, _LA)], pa, sa0),
                    pltpu.make_async_copy(
                        t_hbm.at[0, pl.ds(g0, _GB), pl.ds(0, _LA)], ta, sa1))

        def copies_b(k):
            g0 = g0_of(k)
            return (pltpu.make_async_copy(
                        o_hbm.at[0, pl.ds(g0, _GB), pl.ds(_SPLIT, _LB1)],
                        pb.at[:, pl.ds(0, _LB1)], sb0),
                    pltpu.make_async_copy(
                        t_hbm.at[0, pl.ds(g0, _GB), pl.ds(_SPLIT, _LB1)],
                        tb.at[:, pl.ds(0, _LB1)], sb1),
                    pltpu.make_async_copy(
                        otail_hbm.at[pl.ds(g0, _GB)],
                        pb.at[:, pl.ds(_LB1, _TAILW)], sb0),
                    pltpu.make_async_copy(
                        ttail_hbm.at[pl.ds(g0, _GB)],
                        tb.at[:, pl.ds(_LB1, _TAILW)], sb1))

        def sum_range4(pbuf, tbuf, g4, start, length, buflen):
            nch = (length + 15) // 16 + 1
            c0 = pl.multiple_of(
                jnp.clip((start // 16) * 16, 0, buflen - nch * 16), 16)
            gs = [g4 + j for j in range(_GG)]

            def load(g, o):
                return pbuf[g, pl.ds(o, 16)] - tbuf[g, pl.ds(o, 16)]

            if length == _DAYS:
                mf0 = jnp.clip(lane + (c0 - start + 1), 0, 1)
                mf0 = mf0.astype(jnp.float32)
                regs = tuple(load(g, c0) * mf0 for g in gs)

                def chunk7(i, rs):
                    for u in range(7):
                        o = pl.multiple_of(c0 + 16 + (i * 7 + u) * 16, 16)
                        rs = tuple(r + load(g, o) for r, g in zip(rs, gs))
                    return rs
                regs = lax.fori_loop(0, (nch - 3) // 7, chunk7, regs)
                for i in (nch - 2, nch - 1):
                    mfr = jnp.clip((start + length - c0 - 16 * i) - lane,
                                   0, 1).astype(jnp.float32)
                    o = pl.multiple_of(c0 + 16 * i, 16)
                    regs = tuple(r + load(g, o) * mfr
                                 for r, g in zip(regs, gs))
                return regs
            def chunk_m(i, rs):
                o = pl.multiple_of(c0 + i * 16, 16)
                a = lane + (c0 + 16 * i - start + 1)
                b = (start + length - c0 - 16 * i) - lane
                mf = jnp.clip(jnp.minimum(a, b), 0, 1).astype(jnp.float32)
                return tuple(r + load(g, o) * mf for r, g in zip(rs, gs))
            return lax.fori_loop(0, nch, chunk_m, (zero16,) * _GG)

        def task_body(k, sq):
            task = wid + _NWORK * k
            validf = jnp.where(task < _NTASK, jnp.float32(1.0),
                               jnp.float32(0.0))
            ca = copies_a(k)
            for h in ca:
                h.wait()
            cb = copies_b(k)
            for h in cb:
                h.start()

            def gauge_a4(gp, sq_in):
                g4 = gp * _GG

                def year_a(y, s_in):
                    regs = sum_range4(pa, ta, g4, y * _DAYS, _DAYS, _LA)
                    for r in regs:
                        s = jnp.sum(r)
                        s_in = s_in + s * s
                    return s_in
                sq_g = lax.fori_loop(0, _YSPLIT, year_a, sq_in)
                r10 = sum_range4(pa, ta, g4, _YSPLIT * _DAYS, _L10A, _LA)
                for j in range(_GG):
                    y10s[g4 + j] = jnp.sum(r10[j])
                return sq_g
            sq_a = lax.fori_loop(0, _GB // _GG, gauge_a4, jnp.float32(0.0))

            for h in cb:
                h.wait()
            na = copies_a(k + 1)
            for h in na:
                h.start()

            def gauge_b4(gp, sq_in):
                g4 = gp * _GG
                r10 = sum_range4(pb, tb, g4, 0, _L10B, _LBPAD)
                for j in range(_GG):
                    s10 = y10s[g4 + j] + jnp.sum(r10[j])
                    sq_in = sq_in + s10 * s10

                def year_b(y, s_in):
                    start = (_YSPLIT + 1 + y) * _DAYS - _SPLIT
                    regs = sum_range4(pb, tb, g4, start, _DAYS, _LBPAD)
                    for r in regs:
                        s = jnp.sum(r)
                        s_in = s_in + s * s
                    return s_in
                return lax.fori_loop(0, _NY - _YSPLIT - 1, year_b, sq_in)
            sq_b = lax.fori_loop(0, _GB // _GG, gauge_b4, jnp.float32(0.0))
            return sq + (sq_a + sq_b) * validf

        for h in copies_a(0):
            h.start()
        sq = lax.fori_loop(0, _TPW, task_body, jnp.float32(0.0))
        for h in copies_a(_TPW):
            h.wait()

        def zero_o(r, c):
            obuf[r] = zero16
            return c
        lax.fori_loop(0, 8, zero_o, 0)
        lane0 = (1 - jnp.clip(lane, 0, 1)).astype(jnp.float32)
        obuf[0] = jnp.full((16,), sq, jnp.float32) * lane0
        pltpu.sync_copy(obuf, out_hbm.at[wid])

    return body(o3, t3, otail, ttail)


def _tc_partial(o3, t3):
    def tcbody(o_ref, t_ref, out_ref):
        d = o_ref[0] - t_ref[0]
        day = lax.broadcasted_iota(jnp.int32, (_NT, _NY), 0) // _DAYS
        yid = lax.broadcasted_iota(jnp.int32, (_NT, _NY), 1)
        ymat = (day == yid).astype(jnp.float32)
        s = lax.dot_general(d, ymat, (((1,), (0,)), ((), ())),
                            preferred_element_type=jnp.float32,
                            precision=lax.Precision.HIGHEST)
        ps = jnp.sum(s * s)

        @pl.when(pl.program_id(0) == 0)
        def _init():
            out_ref[...] = jnp.zeros((1, 1), jnp.float32)
        out_ref[...] += jnp.full((1, 1), ps, jnp.float32)

    return pl.pallas_call(
        tcbody,
        grid=(_NGTC // _GTC,),
        in_specs=[pl.BlockSpec((1, _GTC, _NT), lambda i: (0, i, 0)),
                  pl.BlockSpec((1, _GTC, _NT), lambda i: (0, i, 0))],
        out_specs=pl.BlockSpec((1, 1), lambda i: (0, 0)),
        out_shape=jax.ShapeDtypeStruct((1, 1), jnp.float32),
    )(o3, t3)


def kernel(output, target):
    o3 = jnp.transpose(output, (2, 1, 0))
    t3 = jnp.transpose(target, (2, 1, 0))
    otail = jnp.pad(o3[0, :, _NT - _NTAIL:], ((0, 0), (0, _TAILW - _NTAIL)))
    ttail = jnp.pad(t3[0, :, _NT - _NTAIL:], ((0, 0), (0, _TAILW - _NTAIL)))
    partials = _sc_partials(o3, t3, otail, ttail)
    tc_part = _tc_partial(o3, t3)
    scale = 1.0 / (float(_DAYS) * float(_DAYS) * float(_NY) * float(_NG))
    return (jnp.sum(partials) + tc_part[0, 0]) * scale


# speedup vs baseline: 1.1536x; 1.1536x over previous
import functools

import jax
import jax.numpy as jnp
from jax import lax
from jax.experimental import pallas as pl
from jax.experimental.pallas import tpu as pltpu
from jax.experimental.pallas import tpu_sc as plsc

_NT = 7300
_NG = 2000
_DAYS = 365
_NY = _NT // _DAYS
_NGTC = 1024
_GTC = 128
_GB = 8
_GG = 4
_NTASK = (_NG - _NGTC) // _GB
_NWORK = 32
_TPW = -(-_NTASK // _NWORK)
_SPLIT = 3712
_LA = _SPLIT
_LB = _NT - _SPLIT
_YSPLIT = _SPLIT // _DAYS
_L10A = _SPLIT - _YSPLIT * _DAYS
_L10B = _DAYS - _L10A
_LB1 = ((_NT // 128) * 128) - _SPLIT
_NTAIL = _NT - _SPLIT - _LB1
_TAILW = 128
_LBPAD = _LB1 + _TAILW


def _sc_partials(o3, t3, otail, ttail):
    mesh = plsc.VectorSubcoreMesh(core_axis_name="c", subcore_axis_name="s")

    @functools.partial(
        pl.kernel,
        out_type=jax.ShapeDtypeStruct((_NWORK, 8, 16), jnp.float32),
        mesh=mesh,
        scratch_types=[
            pltpu.VMEM((_GB, _LA), jnp.float32),
            pltpu.VMEM((_GB, _LA), jnp.float32),
            pltpu.VMEM((_GB, _LBPAD), jnp.float32),
            pltpu.VMEM((_GB, _LBPAD), jnp.float32),
            pltpu.VMEM((8, 16), jnp.float32),
            pltpu.SMEM((_GB,), jnp.float32),
            pltpu.SemaphoreType.DMA,
            pltpu.SemaphoreType.DMA,
            pltpu.SemaphoreType.DMA,
            pltpu.SemaphoreType.DMA,
        ],
        compiler_params=pltpu.CompilerParams(needs_layout_passes=False),
    )
    def body(o_hbm, t_hbm, otail_hbm, ttail_hbm, out_hbm, pa, ta, pb, tb,
             obuf, y10s, sa0, sa1, sb0, sb1):
        wid = lax.axis_index("s") * 2 + lax.axis_index("c")
        lane = lax.broadcasted_iota(jnp.int32, (16,), 0)
        zero16 = jnp.zeros((16,), jnp.float32)

        def g0_of(k):
            return pl.multiple_of(
                jnp.minimum(wid + _NWORK * k, _NTASK - 1) * _GB + _NGTC, _GB)

        def copies_a(k):
            g0 = g0_of(k)
            return (pltpu.make_async_copy(
                        o_hbm.at[0, pl.ds(g0, _GB), pl.ds(0, _LA)], pa, sa0),
                    pltpu.make_async_copy(
                        t_hbm.at[0, pl.ds(g0, _GB), pl.ds(0, _LA)], ta, sa1))

        def copies_b(k):
            g0 = g0_of(k)
            return (pltpu.make_async_copy(
                        o_hbm.at[0, pl.ds(g0, _GB), pl.ds(_SPLIT, _LB1)],
                        pb.at[:, pl.ds(0, _LB1)], sb0),
                    pltpu.make_async_copy(
                        t_hbm.at[0, pl.ds(g0, _GB), pl.ds(_SPLIT, _LB1)],
                        tb.at[:, pl.ds(0, _LB1)], sb1),
                    pltpu.make_async_copy(
                        otail_hbm.at[pl.ds(g0, _GB)],
                        pb.at[:, pl.ds(_LB1, _TAILW)], sb0),
                    pltpu.make_async_copy(
                        ttail_hbm.at[pl.ds(g0, _GB)],
                        tb.at[:, pl.ds(_LB1, _TAILW)], sb1))

        def sum_range4(pbuf, tbuf, g4, start, length, buflen):
            nch = (length + 15) // 16 + 1
            c0 = pl.multiple_of(
                jnp.clip((start // 16) * 16, 0, buflen - nch * 16), 16)
            gs = [g4 + j for j in range(_GG)]

            def load(g, o):
                return pbuf[g, pl.ds(o, 16)] - tbuf[g, pl.ds(o, 16)]

            if length == _DAYS:
                mf0 = jnp.clip(lane + (c0 - start + 1), 0, 1)
                mf0 = mf0.astype(jnp.float32)
                regs = tuple(load(g, c0) * mf0 for g in gs)

                def chunk7(i, rs):
                    for u in range(7):
                        o = pl.multiple_of(c0 + 16 + (i * 7 + u) * 16, 16)
                        rs = tuple(r + load(g, o) for r, g in zip(rs, gs))
                    return rs
                regs = lax.fori_loop(0, (nch - 3) // 7, chunk7, regs)
                for i in (nch - 2, nch - 1):
                    mfr = jnp.clip((start + length - c0 - 16 * i) - lane,
                                   0, 1).astype(jnp.float32)
                    o = pl.multiple_of(c0 + 16 * i, 16)
                    regs = tuple(r + load(g, o) * mfr
                                 for r, g in zip(regs, gs))
                return regs
            def chunk_m(i, rs):
                o = pl.multiple_of(c0 + i * 16, 16)
                a = lane + (c0 + 16 * i - start + 1)
                b = (start + length - c0 - 16 * i) - lane
                mf = jnp.clip(jnp.minimum(a, b), 0, 1).astype(jnp.float32)
                return tuple(r + load(g, o) * mf for r, g in zip(rs, gs))
            return lax.fori_loop(0, nch, chunk_m, (zero16,) * _GG)

        def task_body(k, sq):
            task = wid + _NWORK * k
            validf = jnp.where(task < _NTASK, jnp.float32(1.0),
                               jnp.float32(0.0))
            ca = copies_a(k)
            for h in ca:
                h.wait()
            cb = copies_b(k)
            for h in cb:
                h.start()

            def gauge_a4(gp, sq_in):
                g4 = gp * _GG

                def year_a(y, s_in):
                    regs = sum_range4(pa, ta, g4, y * _DAYS, _DAYS, _LA)
                    for r in regs:
                        s = jnp.sum(r)
                        s_in = s_in + s * s
                    return s_in
                sq_g = lax.fori_loop(0, _YSPLIT, year_a, sq_in)
                r10 = sum_range4(pa, ta, g4, _YSPLIT * _DAYS, _L10A, _LA)
                for j in range(_GG):
                    y10s[g4 + j] = jnp.sum(r10[j])
                return sq_g
            sq_a = lax.fori_loop(0, _GB // _GG, gauge_a4, jnp.float32(0.0))

            for h in cb:
                h.wait()
            na = copies_a(k + 1)
            for h in na:
                h.start()

            def gauge_b4(gp, sq_in):
                g4 = gp * _GG
                r10 = sum_range4(pb, tb, g4, 0, _L10B, _LBPAD)
                for j in range(_GG):
                    s10 = y10s[g4 + j] + jnp.sum(r10[j])
                    sq_in = sq_in + s10 * s10

                def year_b(y, s_in):
                    start = (_YSPLIT + 1 + y) * _DAYS - _SPLIT
                    regs = sum_range4(pb, tb, g4, start, _DAYS, _LBPAD)
                    for r in regs:
                        s = jnp.sum(r)
                        s_in = s_in + s * s
                    return s_in
                return lax.fori_loop(0, _NY - _YSPLIT - 1, year_b, sq_in)
            sq_b = lax.fori_loop(0, _GB // _GG, gauge_b4, jnp.float32(0.0))
            return sq + (sq_a + sq_b) * validf

        for h in copies_a(0):
            h.start()
        sq = lax.fori_loop(0, _TPW, task_body, jnp.float32(0.0))
        for h in copies_a(_TPW):
            h.wait()

        def zero_o(r, c):
            obuf[r] = zero16
            return c
        lax.fori_loop(0, 8, zero_o, 0)
        lane0 = (1 - jnp.clip(lane, 0, 1)).astype(jnp.float32)
        obuf[0] = jnp.full((16,), sq, jnp.float32) * lane0
        pltpu.sync_copy(obuf, out_hbm.at[wid])

    return body(o3, t3, otail, ttail)


def _tc_partial(o3, t3):
    def tcbody(o_ref, t_ref, out_ref):
        d = o_ref[0] - t_ref[0]
        day = lax.broadcasted_iota(jnp.int32, (_NT, _NY), 0) // _DAYS
        yid = lax.broadcasted_iota(jnp.int32, (_NT, _NY), 1)
        ymat = (day == yid).astype(jnp.float32)
        s = lax.dot_general(d, ymat, (((1,), (0,)), ((), ())),
                            preferred_element_type=jnp.float32,
                            precision=lax.Precision.HIGHEST)
        ps = jnp.sum(s * s)

        @pl.when(pl.program_id(0) == 0)
        def _init():
            out_ref[...] = jnp.zeros((1, 1), jnp.float32)
        out_ref[...] += jnp.full((1, 1), ps, jnp.float32)

    return pl.pallas_call(
        tcbody,
        grid=(_NGTC // _GTC,),
        in_specs=[pl.BlockSpec((1, _GTC, _NT), lambda i: (0, i, 0)),
                  pl.BlockSpec((1, _GTC, _NT), lambda i: (0, i, 0))],
        out_specs=pl.BlockSpec((1, 1), lambda i: (0, 0)),
        out_shape=jax.ShapeDtypeStruct((1, 1), jnp.float32),
    )(o3, t3)


def kernel(output, target):
    o3 = jnp.transpose(output, (2, 1, 0))
    t3 = jnp.transpose(target, (2, 1, 0))
    otail = jnp.pad(o3[0, :, _NT - _NTAIL:], ((0, 0), (0, _TAILW - _NTAIL)))
    ttail = jnp.pad(t3[0, :, _NT - _NTAIL:], ((0, 0), (0, _TAILW - _NTAIL)))
    partials = _sc_partials(o3, t3, otail, ttail)
    tc_part = _tc_partial(o3, t3)
    scale = 1.0 / (float(_DAYS) * float(_DAYS) * float(_NY) * float(_NG))
    return (jnp.sum(partials) + tc_part[0, 0]) * scale
